# baseline (speedup 1.0000x reference)
import functools

import jax
import jax.numpy as jnp
from jax import lax
from jax.experimental import pallas as pl
from jax.experimental.pallas import tpu as pltpu
from jax.experimental.pallas import tpu_sc as plsc

_D = 64
_C = 128


@functools.cache
def _make_gather(n_total: int, n_rows: int):
    info = plsc.get_sparse_core_info()
    nc, ns = info.num_cores, info.num_subcores
    nw = nc * ns
    n_per_w = n_total // nw
    n_chunks = n_per_w // _C
    assert n_per_w * nw == n_total and n_chunks * _C == n_per_w
    assert n_chunks % 2 == 0
    mesh = plsc.VectorSubcoreMesh(core_axis_name="c", subcore_axis_name="s")

    @functools.partial(
        pl.kernel,
        out_type=jax.ShapeDtypeStruct((n_total, _D), jnp.float32),
        mesh=mesh,
        compiler_params=pltpu.CompilerParams(use_tc_tiling_on_sc=False),
        scratch_types=[
            pltpu.VMEM((n_chunks, _C), jnp.int32),
            pltpu.VMEM((_C, _D), jnp.float32),
            pltpu.VMEM((_C, _D), jnp.float32),
            pltpu.SemaphoreType.DMA,
            pltpu.SemaphoreType.DMA,
        ],
    )
    def gather_kernel(table_hbm, idx_hbm, out_hbm, idx_v, buf0, buf1, sem0, sem1):
        wid = lax.axis_index("s") * nc + lax.axis_index("c")
        base = wid * n_per_w
        pltpu.sync_copy(idx_hbm.at[wid], idx_v)

        def start_gather(j, buf, sem):
            pltpu.async_copy(table_hbm.at[idx_v.at[j]], buf, sem)

        def wait_gather(buf, sem):
            pltpu.make_async_copy(table_hbm.at[idx_v.at[0]], buf, sem).wait()

        def put(j, buf):
            pltpu.sync_copy(buf, out_hbm.at[pl.ds(base + j * _C, _C)])

        start_gather(0, buf0, sem0)

        def body(i, carry):
            j0 = 2 * i
            wait_gather(buf0, sem0)
            start_gather(j0 + 1, buf1, sem1)
            put(j0, buf0)
            wait_gather(buf1, sem1)
            start_gather(j0 + 2, buf0, sem0)
            put(j0 + 1, buf1)
            return carry

        lax.fori_loop(0, n_chunks // 2 - 1, body, 0)
        j = n_chunks - 2
        wait_gather(buf0, sem0)
        start_gather(j + 1, buf1, sem1)
        put(j, buf0)
        wait_gather(buf1, sem1)
        put(j + 1, buf1)

    return gather_kernel


def kernel(token_ids, embedding_weight):
    b, s = token_ids.shape
    n = b * s
    info = plsc.get_sparse_core_info()
    nw = info.num_cores * info.num_subcores
    idx = token_ids.reshape(-1).astype(jnp.int32)
    idx3 = idx.reshape(nw, (n // nw) // _C, _C)
    out = _make_gather(n, embedding_weight.shape[0])(embedding_weight, idx3)
    return out.reshape(b, s, _D)
